# baseline (speedup 1.0000x reference)
import functools

import jax
import jax.numpy as jnp
from jax import lax
from jax.experimental import pallas as pl
from jax.experimental.pallas import tpu as pltpu
from jax.experimental.pallas import tpu_sc as plsc

N = 8388608
C = 100000
CROW = 102400
EPS = 1e-7

NC = 2
NS = 16
NW = NC * NS
PER_W = N // NW
CHUNK = 4096
NCHUNK = PER_W // CHUNK
GROUPS = CHUNK // 16
CHUNK_H = 8192
NCHUNK_H = PER_W // CHUNK_H
GROUPS_H = CHUNK_H // 16
LANES = 16

_mesh = plsc.VectorSubcoreMesh(core_axis_name="c", subcore_axis_name="s")
_sc_params = pltpu.CompilerParams(needs_layout_passes=False)


def _worker_id():
  return lax.axis_index("s") * NC + lax.axis_index("c")


@functools.partial(
    pl.kernel,
    out_type=jax.ShapeDtypeStruct((NW * CROW,), jnp.int32),
    mesh=_mesh,
    compiler_params=_sc_params,
    scratch_types=[
        pltpu.VMEM((CHUNK_H,), jnp.int32),
        pltpu.VMEM((CHUNK_H,), jnp.int32),
        pltpu.VMEM((C,), jnp.int32),
        pltpu.SemaphoreType.DMA((2,)),
    ],
)
def _hist_kernel(x_hbm, part_hbm, inbuf0, inbuf1, hist, sem_in):
  inbuf = (inbuf0, inbuf1)
  wid = _worker_id()
  base = pl.multiple_of(wid * PER_W, CHUNK_H)

  def start_in(j, b):
    pltpu.async_copy(
        x_hbm.at[pl.ds(base + j * CHUNK_H, CHUNK_H)], inbuf[b], sem_in.at[b]
    )

  def wait_in(b):
    pltpu.make_async_copy(
        x_hbm.at[pl.ds(base, CHUNK_H)], inbuf[b], sem_in.at[b]
    ).wait()

  for b in range(2):
    start_in(b, b)

  zeros = jnp.zeros((LANES,), jnp.int32)

  @plsc.parallel_loop(0, C // LANES, unroll=8)
  def _zero(i):
    hist[pl.ds(i * LANES, LANES)] = zeros

  def process(b):
    buf = inbuf[b]

    ones = jnp.ones((LANES,), jnp.int32)

    @plsc.parallel_loop(0, GROUPS_H, unroll=16)
    def _grp(g):
      idx = buf[pl.ds(g * LANES, LANES)]
      plsc.addupdate_scatter(hist, [idx], ones)

  def outer(jj, _):
    for b in range(2):
      j = jj * 2 + b
      wait_in(b)
      process(b)
      start_in(j + 2, b)
    return 0

  lax.fori_loop(0, NCHUNK_H // 2 - 1, outer, 0)

  for b in range(2):
    wait_in(b)
    process(b)

  pltpu.sync_copy(hist, part_hbm.at[pl.ds(wid * CROW, C)])


def _inv_body(part_ref, inv_ref):
  i = pl.program_id(0)

  @pl.when(i == 0)
  def _():
    inv_ref[...] = part_ref[...].astype(jnp.float32)

  @pl.when(i > 0)
  def _():
    inv_ref[...] = inv_ref[...] + part_ref[...].astype(jnp.float32)

  @pl.when(i == NW - 1)
  def _():
    inv_ref[...] = 1.0 / jnp.maximum(inv_ref[...], EPS)


_inv_call = pl.pallas_call(
    _inv_body,
    grid=(NW,),
    in_specs=[pl.BlockSpec((CROW,), lambda i: (i,))],
    out_specs=pl.BlockSpec((CROW,), lambda i: (0,)),
    out_shape=jax.ShapeDtypeStruct((CROW,), jnp.float32),
)


CHUNK_I = 8192
NCHUNK_I = PER_W // CHUNK_I
HGROUPS = CHUNK // 16


@functools.partial(
    pl.kernel,
    out_type=jax.ShapeDtypeStruct((N,), jnp.float32),
    mesh=_mesh,
    compiler_params=_sc_params,
    scratch_types=[
        pltpu.VMEM((C,), jnp.float32),
        pltpu.VMEM((CHUNK_I,), jnp.int32),
        pltpu.VMEM((CHUNK_I,), jnp.int32),
        pltpu.VMEM((CHUNK,), jnp.float32),
        pltpu.VMEM((CHUNK,), jnp.float32),
        pltpu.VMEM_SHARED((C,), jnp.float32),
        pltpu.SemaphoreType.DMA((2,)),
        pltpu.SemaphoreType.DMA((2,)),
    ],
)
def _gather_kernel(inv_hbm, x_hbm, out_hbm, table, inbuf0, inbuf1, outbuf0,
                   outbuf1, shared_tab, sem_in, sem_out):
  inbuf = (inbuf0, inbuf1)
  outbuf = (outbuf0, outbuf1)
  wid = _worker_id()
  base = pl.multiple_of(wid * PER_W, CHUNK_I)

  def start_in(j, b):
    pltpu.async_copy(
        x_hbm.at[pl.ds(base + j * CHUNK_I, CHUNK_I)], inbuf[b], sem_in.at[b]
    )

  def wait_in(b):
    pltpu.make_async_copy(
        x_hbm.at[pl.ds(base, CHUNK_I)], inbuf[b], sem_in.at[b]
    ).wait()

  def start_out(jh, h):
    pltpu.async_copy(
        outbuf[h], out_hbm.at[pl.ds(base + jh * CHUNK, CHUNK)], sem_out.at[h]
    )

  def wait_out(h):
    pltpu.make_async_copy(
        outbuf[h], out_hbm.at[pl.ds(base, CHUNK)], sem_out.at[h]
    ).wait()

  def process_half(b, h):
    ibuf = inbuf[b]
    obuf = outbuf[h]

    @plsc.parallel_loop(0, HGROUPS, unroll=16)
    def _grp(g):
      idx = ibuf[pl.ds((h * HGROUPS + g) * LANES, LANES)]
      obuf[pl.ds(g * LANES, LANES)] = plsc.load_gather(table, [idx])

  for b in range(2):
    start_in(b, b)

  @pl.when(lax.axis_index("s") == 0)
  def _():
    pltpu.sync_copy(inv_hbm, shared_tab)

  plsc.subcore_barrier()
  pltpu.sync_copy(shared_tab, table)

  wait_in(0)
  for h in range(2):
    process_half(0, h)
    start_out(h, h)
  start_in(2, 0)

  def outer(jj, _):
    for b in (1, 0):
      j = 2 * jj + 1 + (1 - b)
      wait_in(b)
      for h in range(2):
        wait_out(h)
        process_half(b, h)
        start_out(2 * j + h, h)
      start_in(j + 2, b)
    return 0

  lax.fori_loop(0, (NCHUNK_I - 4) // 2, outer, 0)

  for j, b, prefetch in ((NCHUNK_I - 3, 1, True), (NCHUNK_I - 2, 0, False),
                         (NCHUNK_I - 1, 1, False)):
    wait_in(b)
    for h in range(2):
      wait_out(h)
      process_half(b, h)
      start_out(2 * j + h, h)
    if prefetch:
      start_in(j + 2, b)

  for h in range(2):
    wait_out(h)


@jax.jit
def kernel(inputs):
  x = inputs.astype(jnp.int32).reshape(-1)
  partials = _hist_kernel(x)
  inv = _inv_call(partials)[:C]
  out = _gather_kernel(inv, x)
  return out[:, None]


# speedup vs baseline: 1.1389x; 1.1389x over previous
import functools

import jax
import jax.numpy as jnp
from jax import lax
from jax.experimental import pallas as pl
from jax.experimental.pallas import tpu as pltpu
from jax.experimental.pallas import tpu_sc as plsc

N = 8388608
C = 100000
EPS = 1e-7

NC = 2
NS = 16
NW = NC * NS
PER_W = N // NW
CHUNK = 4096
NCHUNK = PER_W // CHUNK
GROUPS = CHUNK // 16
CHUNK_H = 8192
NCHUNK_H = PER_W // CHUNK_H
GROUPS_H = CHUNK_H // 16
LANES = 16

_mesh = plsc.VectorSubcoreMesh(core_axis_name="c", subcore_axis_name="s")
_sc_params = pltpu.CompilerParams(needs_layout_passes=False)


def _worker_id():
  return lax.axis_index("s") * NC + lax.axis_index("c")


@functools.partial(
    pl.kernel,
    out_type=jax.ShapeDtypeStruct((NW, C), jnp.int32),
    mesh=_mesh,
    compiler_params=_sc_params,
    scratch_types=[
        pltpu.VMEM((CHUNK_H,), jnp.int32),
        pltpu.VMEM((CHUNK_H,), jnp.int32),
        pltpu.VMEM((C,), jnp.int32),
        pltpu.SemaphoreType.DMA((2,)),
    ],
)
def _hist_kernel(x_hbm, part_hbm, inbuf0, inbuf1, hist, sem_in):
  inbuf = (inbuf0, inbuf1)
  wid = _worker_id()
  base = pl.multiple_of(wid * PER_W, CHUNK_H)

  def start_in(j, b):
    pltpu.async_copy(
        x_hbm.at[pl.ds(base + j * CHUNK_H, CHUNK_H)], inbuf[b], sem_in.at[b]
    )

  def wait_in(b):
    pltpu.make_async_copy(
        x_hbm.at[pl.ds(base, CHUNK_H)], inbuf[b], sem_in.at[b]
    ).wait()

  for b in range(2):
    start_in(b, b)

  zeros = jnp.zeros((LANES,), jnp.int32)

  @plsc.parallel_loop(0, C // LANES, unroll=8)
  def _zero(i):
    hist[pl.ds(i * LANES, LANES)] = zeros

  def process(b):
    buf = inbuf[b]

    ones = jnp.ones((LANES,), jnp.int32)

    @plsc.parallel_loop(0, GROUPS_H, unroll=16)
    def _grp(g):
      idx = buf[pl.ds(g * LANES, LANES)]
      plsc.addupdate_scatter(hist, [idx], ones)

  def outer(jj, _):
    for b in range(2):
      j = jj * 2 + b
      wait_in(b)
      process(b)
      start_in(j + 2, b)
    return 0

  lax.fori_loop(0, NCHUNK_H // 2 - 1, outer, 0)

  for b in range(2):
    wait_in(b)
    process(b)

  pltpu.sync_copy(hist, part_hbm.at[wid])


def _inv_body(part_ref, inv_ref):
  counts = jnp.sum(part_ref[...], axis=0).astype(jnp.float32)
  inv_ref[...] = 1.0 / jnp.maximum(counts, EPS)


_inv_call = pl.pallas_call(
    _inv_body,
    out_shape=jax.ShapeDtypeStruct((C,), jnp.float32),
)


CHUNK_I = 8192
NCHUNK_I = PER_W // CHUNK_I
HGROUPS = CHUNK // 16


@functools.partial(
    pl.kernel,
    out_type=jax.ShapeDtypeStruct((N,), jnp.float32),
    mesh=_mesh,
    compiler_params=_sc_params,
    scratch_types=[
        pltpu.VMEM((C,), jnp.float32),
        pltpu.VMEM((CHUNK_I,), jnp.int32),
        pltpu.VMEM((CHUNK_I,), jnp.int32),
        pltpu.VMEM((CHUNK,), jnp.float32),
        pltpu.VMEM((CHUNK,), jnp.float32),
        pltpu.VMEM_SHARED((C,), jnp.float32),
        pltpu.SemaphoreType.DMA((2,)),
        pltpu.SemaphoreType.DMA((2,)),
    ],
)
def _gather_kernel(inv_hbm, x_hbm, out_hbm, table, inbuf0, inbuf1, outbuf0,
                   outbuf1, shared_tab, sem_in, sem_out):
  inbuf = (inbuf0, inbuf1)
  outbuf = (outbuf0, outbuf1)
  wid = _worker_id()
  base = pl.multiple_of(wid * PER_W, CHUNK_I)

  def start_in(j, b):
    pltpu.async_copy(
        x_hbm.at[pl.ds(base + j * CHUNK_I, CHUNK_I)], inbuf[b], sem_in.at[b]
    )

  def wait_in(b):
    pltpu.make_async_copy(
        x_hbm.at[pl.ds(base, CHUNK_I)], inbuf[b], sem_in.at[b]
    ).wait()

  def start_out(jh, h):
    pltpu.async_copy(
        outbuf[h], out_hbm.at[pl.ds(base + jh * CHUNK, CHUNK)], sem_out.at[h]
    )

  def wait_out(h):
    pltpu.make_async_copy(
        outbuf[h], out_hbm.at[pl.ds(base, CHUNK)], sem_out.at[h]
    ).wait()

  def process_half(b, h):
    ibuf = inbuf[b]
    obuf = outbuf[h]

    @plsc.parallel_loop(0, HGROUPS, unroll=16)
    def _grp(g):
      idx = ibuf[pl.ds((h * HGROUPS + g) * LANES, LANES)]
      obuf[pl.ds(g * LANES, LANES)] = plsc.load_gather(table, [idx])

  for b in range(2):
    start_in(b, b)

  @pl.when(lax.axis_index("s") == 0)
  def _():
    pltpu.sync_copy(inv_hbm, shared_tab)

  plsc.subcore_barrier()
  pltpu.sync_copy(shared_tab, table)

  wait_in(0)
  for h in range(2):
    process_half(0, h)
    start_out(h, h)
  start_in(2, 0)

  def outer(jj, _):
    for b in (1, 0):
      j = 2 * jj + 1 + (1 - b)
      wait_in(b)
      for h in range(2):
        wait_out(h)
        process_half(b, h)
        start_out(2 * j + h, h)
      start_in(j + 2, b)
    return 0

  lax.fori_loop(0, (NCHUNK_I - 4) // 2, outer, 0)

  for j, b, prefetch in ((NCHUNK_I - 3, 1, True), (NCHUNK_I - 2, 0, False),
                         (NCHUNK_I - 1, 1, False)):
    wait_in(b)
    for h in range(2):
      wait_out(h)
      process_half(b, h)
      start_out(2 * j + h, h)
    if prefetch:
      start_in(j + 2, b)

  for h in range(2):
    wait_out(h)


@jax.jit
def kernel(inputs):
  x = inputs.astype(jnp.int32).reshape(-1)
  partials = _hist_kernel(x)
  inv = _inv_call(partials)
  out = _gather_kernel(inv, x)
  return out[:, None]
